# baseline (speedup 1.0000x reference)
import jax
import jax.numpy as jnp
from jax import lax
from jax.experimental import pallas as pl
from jax.experimental.pallas import tpu as pltpu


def _two_layer_cheb_kernel(x_ref, a_ref, w1_ref, b1_ref, w2_ref, b2_ref,
                           lsm_ref, out_ref):
    xg = x_ref[0]
    ag = a_ref[0]
    n = ag.shape[0]

    deg = jnp.sum(ag, axis=1, keepdims=True)
    dinv = jnp.where(deg > 0, lax.rsqrt(deg), 0.0)
    rows = lax.broadcasted_iota(jnp.int32, (n, n), 0)
    cols = lax.broadcasted_iota(jnp.int32, (n, n), 1)
    diag = jnp.sum(jnp.where(rows == cols, ag, 0.0), axis=1,
                   keepdims=True)

    a_bf = ag.astype(jnp.bfloat16)

    def propq(v):
        y = dinv * v
        y_hi = y.astype(jnp.bfloat16)
        y_lo = (y - y_hi.astype(jnp.float32)).astype(jnp.bfloat16)
        dims = (((0,), (0,)), ((), ()))
        z = lax.dot_general(a_bf, y_hi, dims,
                            preferred_element_type=jnp.float32)
        z += lax.dot_general(a_bf, y_lo, dims,
                             preferred_element_type=jnp.float32)
        return dinv * z + diag * v

    def cheb(v, w_ref, bias_ref):
        q1 = propq(v)
        q2 = propq(q1)
        out = jnp.dot(v, w_ref[0] - w_ref[2],
                      preferred_element_type=jnp.float32)
        out -= jnp.dot(q1, w_ref[1], preferred_element_type=jnp.float32)
        out += jnp.dot(q2, 2.0 * w_ref[2],
                       preferred_element_type=jnp.float32)
        return out + bias_ref[0]

    h = jax.nn.relu(cheb(xg, w1_ref, b1_ref))
    out = cheb(h, w2_ref, b2_ref)

    m = jnp.max(out, axis=1, keepdims=True)
    e = jnp.exp(out - m)
    lse = m + jnp.log(jnp.sum(e, axis=1, keepdims=True))

    out_ref[0] = out
    lsm_ref[0] = out - lse


def kernel(x, A, W1, b1, W2, b2):
    b, n, din = x.shape
    dh = W1.shape[2]
    dout = W2.shape[2]
    K = W1.shape[0]

    b1r = b1.reshape(1, dh)
    b2r = b2.reshape(1, dout)

    lsm, out = pl.pallas_call(
        _two_layer_cheb_kernel,
        grid=(b,),
        in_specs=[
            pl.BlockSpec((1, n, din), lambda i: (i, 0, 0)),
            pl.BlockSpec((1, n, n), lambda i: (i, 0, 0)),
            pl.BlockSpec((K, din, dh), lambda i: (0, 0, 0)),
            pl.BlockSpec((1, dh), lambda i: (0, 0)),
            pl.BlockSpec((K, dh, dout), lambda i: (0, 0, 0)),
            pl.BlockSpec((1, dout), lambda i: (0, 0)),
        ],
        out_specs=[
            pl.BlockSpec((1, n, dout), lambda i: (i, 0, 0)),
            pl.BlockSpec((1, n, dout), lambda i: (i, 0, 0)),
        ],
        out_shape=[
            jax.ShapeDtypeStruct((b, n, dout), jnp.float32),
            jax.ShapeDtypeStruct((b, n, dout), jnp.float32),
        ],
        compiler_params=pltpu.CompilerParams(
            dimension_semantics=("parallel",),
        ),
    )(x, A, W1, b1r, W2, b2r)
    return (lsm, out)


# speedup vs baseline: 1.2476x; 1.2476x over previous
import jax
import jax.numpy as jnp
from jax import lax
from jax.experimental import pallas as pl
from jax.experimental.pallas import tpu as pltpu


def _two_layer_cheb_kernel(x_ref, a_ref, w1_ref, b1_ref, w2_ref, b2_ref,
                           lsm_ref, out_ref):
    b = x_ref.shape[0]
    n = a_ref.shape[1]

    for g in range(b):
        xg = x_ref[g]
        ag = a_ref[g]

        deg = jnp.sum(ag, axis=1, keepdims=True)
        dinv = jnp.where(deg > 0, lax.rsqrt(deg), 0.0)
        rows = lax.broadcasted_iota(jnp.int32, (n, n), 0)
        cols = lax.broadcasted_iota(jnp.int32, (n, n), 1)
        diag = jnp.sum(jnp.where(rows == cols, ag, 0.0), axis=1,
                       keepdims=True)

        def propq(v):
            z = lax.dot_general(ag, dinv * v,
                                (((0,), (0,)), ((), ())),
                                preferred_element_type=jnp.float32)
            return dinv * z + diag * v

        def cheb(v, w_ref, bias_ref):
            q1 = propq(v)
            q2 = propq(q1)
            out = jnp.dot(v, w_ref[0] - w_ref[2],
                          preferred_element_type=jnp.float32)
            out -= jnp.dot(q1, w_ref[1], preferred_element_type=jnp.float32)
            out += jnp.dot(q2, 2.0 * w_ref[2],
                           preferred_element_type=jnp.float32)
            return out + bias_ref[0]

        h = jax.nn.relu(cheb(xg, w1_ref, b1_ref))
        out = cheb(h, w2_ref, b2_ref)

        m = jnp.max(out, axis=1, keepdims=True)
        e = jnp.exp(out - m)
        lse = m + jnp.log(jnp.sum(e, axis=1, keepdims=True))

        out_ref[g] = out
        lsm_ref[g] = out - lse


def kernel(x, A, W1, b1, W2, b2):
    b, n, din = x.shape
    dh = W1.shape[2]
    dout = W2.shape[2]

    b1r = b1.reshape(1, dh)
    b2r = b2.reshape(1, dout)

    lsm, out = pl.pallas_call(
        _two_layer_cheb_kernel,
        out_shape=[
            jax.ShapeDtypeStruct((b, n, dout), jnp.float32),
            jax.ShapeDtypeStruct((b, n, dout), jnp.float32),
        ],
    )(x, A, W1, b1r, W2, b2r)
    return (lsm, out)


# speedup vs baseline: 1.2882x; 1.0326x over previous
import jax
import jax.numpy as jnp
from jax import lax
from jax.experimental import pallas as pl
from jax.experimental.pallas import tpu as pltpu


def _two_layer_cheb_kernel(x_ref, a_ref, w1_ref, b1_ref, w2_ref, b2_ref,
                           lsm_ref, out_ref):
    b = x_ref.shape[0]
    n = a_ref.shape[1]

    for g in range(b):
        xg = x_ref[g]
        ag = a_ref[g]

        deg = jnp.sum(ag, axis=1, keepdims=True)
        dinv = jnp.where(deg > 0, lax.rsqrt(deg), 0.0)
        rows = lax.broadcasted_iota(jnp.int32, (n, n), 0)
        cols = lax.broadcasted_iota(jnp.int32, (n, n), 1)
        diag = jnp.sum(jnp.where(rows == cols, ag, 0.0), axis=1,
                       keepdims=True)

        def propq(v):
            z = lax.dot_general(ag, dinv * v,
                                (((0,), (0,)), ((), ())),
                                preferred_element_type=jnp.float32)
            return dinv * z + diag * v

        def cheb(v, w_ref, bias_ref):
            q1 = propq(v)
            q2 = propq(q1)
            out = jnp.dot(v, w_ref[0] - w_ref[2],
                          preferred_element_type=jnp.float32)
            out -= jnp.dot(q1, w_ref[1], preferred_element_type=jnp.float32)
            out += jnp.dot(q2, 2.0 * w_ref[2],
                           preferred_element_type=jnp.float32)
            return out + bias_ref[0]

        h = jax.nn.relu(cheb(xg, w1_ref, b1_ref))
        out = cheb(h, w2_ref, b2_ref)

        m = jnp.max(out, axis=1, keepdims=True)
        e = jnp.exp(out - m)
        lse = m + jnp.log(jnp.sum(e, axis=1, keepdims=True))

        out_ref[g] = out
        lsm_ref[g] = out - lse


def kernel(x, A, W1, b1, W2, b2):
    b, n, din = x.shape
    dh = W1.shape[2]
    dout = W2.shape[2]
    K = W1.shape[0]

    b1r = b1.reshape(1, dh)
    b2r = b2.reshape(1, dout)

    gb = 2 if b % 2 == 0 else 1
    lsm, out = pl.pallas_call(
        _two_layer_cheb_kernel,
        grid=(b // gb,),
        in_specs=[
            pl.BlockSpec((gb, n, din), lambda i: (i, 0, 0)),
            pl.BlockSpec((gb, n, n), lambda i: (i, 0, 0)),
            pl.BlockSpec((K, din, dh), lambda i: (0, 0, 0)),
            pl.BlockSpec((1, dh), lambda i: (0, 0)),
            pl.BlockSpec((K, dh, dout), lambda i: (0, 0, 0)),
            pl.BlockSpec((1, dout), lambda i: (0, 0)),
        ],
        out_specs=[
            pl.BlockSpec((gb, n, dout), lambda i: (i, 0, 0)),
            pl.BlockSpec((gb, n, dout), lambda i: (i, 0, 0)),
        ],
        out_shape=[
            jax.ShapeDtypeStruct((b, n, dout), jnp.float32),
            jax.ShapeDtypeStruct((b, n, dout), jnp.float32),
        ],
        compiler_params=pltpu.CompilerParams(
            dimension_semantics=("parallel",),
        ),
    )(x, A, W1, b1r, W2, b2r)
    return (lsm, out)


# speedup vs baseline: 1.2923x; 1.0032x over previous
import jax
import jax.numpy as jnp
from jax import lax
from jax.experimental import pallas as pl
from jax.experimental.pallas import tpu as pltpu


def _two_layer_cheb_kernel(x_ref, a_ref, w1_ref, b1_ref, w2_ref, b2_ref,
                           lsm_ref, out_ref):
    b = x_ref.shape[0]
    n = a_ref.shape[1]

    for g in range(b):
        xg = x_ref[g]
        ag = a_ref[g]

        deg = jnp.sum(ag, axis=1, keepdims=True)
        dinv = jnp.where(deg > 0, lax.rsqrt(deg), 0.0)
        rows = lax.broadcasted_iota(jnp.int32, (n, n), 0)
        cols = lax.broadcasted_iota(jnp.int32, (n, n), 1)
        diag = jnp.sum(jnp.where(rows == cols, ag, 0.0), axis=1,
                       keepdims=True)

        def prop(v):
            z = lax.dot_general(ag, dinv * v,
                                (((0,), (0,)), ((), ())),
                                preferred_element_type=jnp.float32)
            return -dinv * z - diag * v

        def cheb(v, w_ref, bias_ref):
            t1 = prop(v)
            t2 = 2.0 * prop(t1) - v
            out = jnp.dot(v, w_ref[0], preferred_element_type=jnp.float32)
            out += jnp.dot(t1, w_ref[1], preferred_element_type=jnp.float32)
            out += jnp.dot(t2, w_ref[2], preferred_element_type=jnp.float32)
            return out + bias_ref[0]

        h = jax.nn.relu(cheb(xg, w1_ref, b1_ref))
        out = cheb(h, w2_ref, b2_ref)

        m = jnp.max(out, axis=1, keepdims=True)
        e = jnp.exp(out - m)
        lse = m + jnp.log(jnp.sum(e, axis=1, keepdims=True))

        out_ref[g] = out
        lsm_ref[g] = out - lse


def kernel(x, A, W1, b1, W2, b2):
    b, n, din = x.shape
    dh = W1.shape[2]
    dout = W2.shape[2]
    K = W1.shape[0]

    b1r = b1.reshape(1, dh)
    b2r = b2.reshape(1, dout)

    gb = 2 if b % 2 == 0 else 1
    lsm, out = pl.pallas_call(
        _two_layer_cheb_kernel,
        grid=(b // gb,),
        in_specs=[
            pl.BlockSpec((gb, n, din), lambda i: (i, 0, 0)),
            pl.BlockSpec((gb, n, n), lambda i: (i, 0, 0)),
            pl.BlockSpec((K, din, dh), lambda i: (0, 0, 0)),
            pl.BlockSpec((1, dh), lambda i: (0, 0)),
            pl.BlockSpec((K, dh, dout), lambda i: (0, 0, 0)),
            pl.BlockSpec((1, dout), lambda i: (0, 0)),
        ],
        out_specs=[
            pl.BlockSpec((gb, n, dout), lambda i: (i, 0, 0)),
            pl.BlockSpec((gb, n, dout), lambda i: (i, 0, 0)),
        ],
        out_shape=[
            jax.ShapeDtypeStruct((b, n, dout), jnp.float32),
            jax.ShapeDtypeStruct((b, n, dout), jnp.float32),
        ],
        compiler_params=pltpu.CompilerParams(
            dimension_semantics=("parallel",),
        ),
    )(x, A, W1, b1r, W2, b2r)
    return (lsm, out)
